# baseline (speedup 1.0000x reference)
import jax
import jax.numpy as jnp
from jax import lax
from jax.experimental import pallas as pl
from jax.experimental.pallas import tpu as pltpu
from jax.experimental.pallas import tpu_sc as plsc

VOCAB = 100000
DIM = 8
KS = 256
SUBDIM = 16
ORIG = DIM * SUBDIM

NC, NS, LANES = 2, 16, 16
NW = NC * NS
N_TOK = 4096 * 50
C = 128
TPW = N_TOK // NW
PAIRS = TPW // (2 * C)


def _pq_decode_body(idx_hbm, indexes_hbm, vectors_hbm, out_hbm,
                    idx0, idx1, codes0, codes1, flat0, flat1, out0, out1,
                    semb0, semb1, semd0, semd1, seme0, seme1):
    wid = lax.axis_index("s") * NC + lax.axis_index("c")
    base = wid * TPW
    idx_v = (idx0, idx1)
    codes_v = (codes0, codes1)
    flat_v = (flat0, flat1)
    out_v = (out0, out1)

    iot = lax.broadcasted_iota(jnp.int32, (16,), 0)

    def fire_ab(t0, b):
        pltpu.sync_copy(idx_hbm.at[pl.ds(t0, C)], idx_v[b])
        pltpu.async_copy(indexes_hbm.at[idx_v[b]], codes_v[b], semb[b])

    def wait_b(b):
        pltpu.make_async_copy(
            indexes_hbm.at[idx_v[b]], codes_v[b], semb[b]).wait()

    def build_and_fire_d(b):
        for d in range(DIM):
            col = jnp.full((16,), d, jnp.int32)
            flat_row = flat_v[b].at[d]

            @pl.loop(0, C // 16)
            def _grp(tt):
                codes16 = plsc.load_gather(codes_v[b], [iot + 16 * tt, col])
                flat_row[pl.ds(16 * tt, 16)] = codes16 + (d << 8)

            pltpu.async_copy(
                vectors_hbm.at[flat_v[b].at[d]],
                out_v[b].at[d], semd[b])

    def wait_d(b):
        for d in range(DIM):
            pltpu.make_async_copy(
                vectors_hbm.at[flat_v[b].at[d]],
                out_v[b].at[d], semd[b]).wait()

    def fire_e(t0, b):
        for d in range(DIM):
            pltpu.async_copy(
                out_v[b].at[d],
                out_hbm.at[pl.ds(t0, C), pl.ds(16 * d, 16)], seme[b])

    def wait_e(b):
        for d in range(DIM):
            pltpu.make_async_copy(
                out_v[b].at[d],
                out_hbm.at[pl.ds(0, C), pl.ds(16 * d, 16)], seme[b]).wait()

    semb = (semb0, semb1)
    semd = (semd0, semd1)
    seme = (seme0, seme1)

    fire_ab(base, 0)
    fire_ab(base + C, 1)

    @pl.loop(0, PAIRS)
    def _pair(g):
        t0 = base + (2 * g) * C
        t1 = t0 + C

        wait_b(0)

        @pl.when(g > 0)
        def _():
            wait_e(0)
        build_and_fire_d(0)

        wait_b(1)

        @pl.when(g > 0)
        def _():
            wait_e(1)
        build_and_fire_d(1)

        @pl.when(g < PAIRS - 1)
        def _():
            fire_ab(t0 + 2 * C, 0)
            fire_ab(t0 + 3 * C, 1)

        wait_d(0)
        fire_e(t0, 0)
        wait_d(1)
        fire_e(t1, 1)

    wait_e(0)
    wait_e(1)


def kernel(idx, indexes, vectors, dims):
    del dims
    idx_flat = idx.reshape(-1)
    vec_flat = vectors.reshape(DIM * KS, SUBDIM)
    mesh = plsc.VectorSubcoreMesh(core_axis_name="c", subcore_axis_name="s")
    cp = pltpu.CompilerParams(
        needs_layout_passes=False, use_tc_tiling_on_sc=False)
    decode = pl.kernel(
        _pq_decode_body,
        out_type=jax.ShapeDtypeStruct((N_TOK, ORIG), jnp.float32),
        mesh=mesh,
        scratch_types=[
            pltpu.VMEM((C,), jnp.int32),
            pltpu.VMEM((C,), jnp.int32),
            pltpu.VMEM((C, DIM), jnp.int32),
            pltpu.VMEM((C, DIM), jnp.int32),
            pltpu.VMEM((DIM, C), jnp.int32),
            pltpu.VMEM((DIM, C), jnp.int32),
            pltpu.VMEM((DIM, C, SUBDIM), jnp.float32),
            pltpu.VMEM((DIM, C, SUBDIM), jnp.float32),
            pltpu.SemaphoreType.DMA,
            pltpu.SemaphoreType.DMA,
            pltpu.SemaphoreType.DMA,
            pltpu.SemaphoreType.DMA,
            pltpu.SemaphoreType.DMA,
            pltpu.SemaphoreType.DMA,
        ],
        compiler_params=cp,
    )
    out = decode(idx_flat, indexes, vec_flat)
    return out.reshape(idx.shape + (ORIG,))


# speedup vs baseline: 1.5699x; 1.5699x over previous
import jax
import jax.numpy as jnp
from jax import lax
from jax.experimental import pallas as pl
from jax.experimental.pallas import tpu as pltpu
from jax.experimental.pallas import tpu_sc as plsc

VOCAB = 100000
DIM = 8
KS = 256
SUBDIM = 16
ORIG = DIM * SUBDIM

BATCH = 4096
HIST = 50
NC, NS, LANES = 2, 16, 16
NW = NC * NS
N_TOK = BATCH * HIST
C = 128
PAIRS = HIST // 2


def _pq_decode_body(idx_hbm, indexes_hbm, vectors_hbm, out_hbm,
                    idx0, idx1, codes0, codes1, flat0, flat1, out0, out1,
                    semb0, semb1, semd0, semd1, seme0, seme1):
    wid = lax.axis_index("s") * NC + lax.axis_index("c")
    base = wid * C
    idx_v = (idx0, idx1)
    codes_v = (codes0, codes1)
    flat_v = (flat0, flat1)
    out_v = (out0, out1)
    semb = (semb0, semb1)
    semd = (semd0, semd1)
    seme = (seme0, seme1)

    iot = lax.broadcasted_iota(jnp.int32, (16,), 0)
    row_pat = iot >> 3
    col_pat = iot & 7
    off_pat = col_pat << 8

    def fire_ab(t0, b):
        pltpu.sync_copy(idx_hbm.at[pl.ds(t0, C)], idx_v[b])
        pltpu.async_copy(indexes_hbm.at[idx_v[b]], codes_v[b], semb[b])

    def wait_b(b):
        pltpu.make_async_copy(
            indexes_hbm.at[idx_v[b]], codes_v[b], semb[b]).wait()

    def build_flat(b):
        @pl.loop(0, C // 2)
        def _grp(g):
            codes16 = plsc.load_gather(codes_v[b], [row_pat + 2 * g, col_pat])
            flat_v[b][pl.ds(16 * g, 16)] = codes16 + off_pat

    def fire_d(b):
        for j in range(DIM):
            pltpu.async_copy(
                vectors_hbm.at[flat_v[b].at[pl.ds(128 * j, 128)]],
                out_v[b].at[pl.ds(128 * j, 128)], semd[b])

    def wait_d(b):
        for j in range(DIM):
            pltpu.make_async_copy(
                vectors_hbm.at[flat_v[b].at[pl.ds(128 * j, 128)]],
                out_v[b].at[pl.ds(128 * j, 128)], semd[b]).wait()

    def fire_e(t0, b):
        pltpu.async_copy(
            out_v[b], out_hbm.at[pl.ds(t0 * DIM, C * DIM)], seme[b])

    def wait_e(b):
        pltpu.make_async_copy(
            out_v[b], out_hbm.at[pl.ds(0, C * DIM)], seme[b]).wait()

    fire_ab(base, 0)
    fire_ab(base + BATCH, 1)

    @pl.loop(0, PAIRS)
    def _pair(g):
        t0 = (2 * g) * BATCH + base
        t1 = t0 + BATCH

        wait_b(0)
        build_flat(0)

        @pl.when(g > 0)
        def _():
            wait_e(0)
        fire_d(0)

        wait_b(1)
        build_flat(1)

        @pl.when(g > 0)
        def _():
            wait_e(1)
        fire_d(1)

        @pl.when(g < PAIRS - 1)
        def _():
            fire_ab(t0 + 2 * BATCH, 0)
            fire_ab(t0 + 3 * BATCH, 1)

        wait_d(0)
        fire_e(t0, 0)
        wait_d(1)
        fire_e(t1, 1)

    wait_e(0)
    wait_e(1)


def kernel(idx, indexes, vectors, dims):
    del dims
    idx_t = jnp.swapaxes(idx, 0, 1).reshape(-1)
    vec_flat = vectors.reshape(DIM * KS, SUBDIM)
    mesh = plsc.VectorSubcoreMesh(core_axis_name="c", subcore_axis_name="s")
    cp = pltpu.CompilerParams(
        needs_layout_passes=False, use_tc_tiling_on_sc=False)
    decode = pl.kernel(
        _pq_decode_body,
        out_type=jax.ShapeDtypeStruct((N_TOK * DIM, SUBDIM), jnp.float32),
        mesh=mesh,
        scratch_types=[
            pltpu.VMEM((C,), jnp.int32),
            pltpu.VMEM((C,), jnp.int32),
            pltpu.VMEM((C, DIM), jnp.int32),
            pltpu.VMEM((C, DIM), jnp.int32),
            pltpu.VMEM((C * DIM,), jnp.int32),
            pltpu.VMEM((C * DIM,), jnp.int32),
            pltpu.VMEM((C * DIM, SUBDIM), jnp.float32),
            pltpu.VMEM((C * DIM, SUBDIM), jnp.float32),
            pltpu.SemaphoreType.DMA,
            pltpu.SemaphoreType.DMA,
            pltpu.SemaphoreType.DMA,
            pltpu.SemaphoreType.DMA,
            pltpu.SemaphoreType.DMA,
            pltpu.SemaphoreType.DMA,
        ],
        compiler_params=cp,
    )
    out = decode(idx_t, indexes, vec_flat)
    out = out.reshape(HIST, BATCH, ORIG)
    return jnp.swapaxes(out, 0, 1)


# speedup vs baseline: 2.9494x; 1.8788x over previous
import jax
import jax.numpy as jnp
from jax import lax
from jax.experimental import pallas as pl
from jax.experimental.pallas import tpu as pltpu
from jax.experimental.pallas import tpu_sc as plsc

VOCAB = 100000
DIM = 8
KS = 256
SUBDIM = 16
ORIG = DIM * SUBDIM

BATCH = 4096
HIST = 50
NC, NS, LANES = 2, 16, 16
NW = NC * NS
N_TOK = BATCH * HIST
C = 128
PAIRS = HIST // 2


def _pq_decode_body(idx_hbm, indexes_hbm, vectors_hbm, out_hbm,
                    idx0, idx1, codes0, codes1, flat0, flat1, out0, out1,
                    vec_sh,
                    semb0, semb1, semd0, semd1, seme0, seme1):
    sid = lax.axis_index("s")
    wid = sid * NC + lax.axis_index("c")
    base = wid * C

    vrows = (DIM * KS) // NS
    pltpu.sync_copy(vectors_hbm.at[pl.ds(sid * vrows, vrows)],
                    vec_sh.at[pl.ds(sid * vrows, vrows)])
    plsc.subcore_barrier()
    idx_v = (idx0, idx1)
    codes_v = (codes0, codes1)
    flat_v = (flat0, flat1)
    out_v = (out0, out1)
    semb = (semb0, semb1)
    semd = (semd0, semd1)
    seme = (seme0, seme1)

    iot = lax.broadcasted_iota(jnp.int32, (16,), 0)
    row_pat = iot >> 3
    col_pat = iot & 7
    off_pat = col_pat << 8

    def fire_ab(t0, b):
        pltpu.sync_copy(idx_hbm.at[pl.ds(t0, C)], idx_v[b])
        pltpu.async_copy(indexes_hbm.at[idx_v[b]], codes_v[b], semb[b])

    def wait_b(b):
        pltpu.make_async_copy(
            indexes_hbm.at[idx_v[b]], codes_v[b], semb[b]).wait()

    def build_flat(b):
        @pl.loop(0, C // 2)
        def _grp(g):
            codes16 = plsc.load_gather(codes_v[b], [row_pat + 2 * g, col_pat])
            flat_v[b][pl.ds(16 * g, 16)] = codes16 + off_pat

    def fire_d(b):
        for j in range(DIM):
            pltpu.async_copy(
                vec_sh.at[flat_v[b].at[pl.ds(128 * j, 128)]],
                out_v[b].at[pl.ds(128 * j, 128)], semd[b])

    def wait_d(b):
        for j in range(DIM):
            pltpu.make_async_copy(
                vec_sh.at[flat_v[b].at[pl.ds(128 * j, 128)]],
                out_v[b].at[pl.ds(128 * j, 128)], semd[b]).wait()

    def fire_e(t0, b):
        pltpu.async_copy(
            out_v[b], out_hbm.at[pl.ds(t0 * DIM, C * DIM)], seme[b])

    def wait_e(b):
        pltpu.make_async_copy(
            out_v[b], out_hbm.at[pl.ds(0, C * DIM)], seme[b]).wait()

    fire_ab(base, 0)
    fire_ab(base + BATCH, 1)

    @pl.loop(0, PAIRS)
    def _pair(g):
        t0 = (2 * g) * BATCH + base
        t1 = t0 + BATCH

        wait_b(0)
        build_flat(0)

        @pl.when(g > 0)
        def _():
            wait_e(0)
        fire_d(0)

        wait_b(1)
        build_flat(1)

        @pl.when(g > 0)
        def _():
            wait_e(1)
        fire_d(1)

        @pl.when(g < PAIRS - 1)
        def _():
            fire_ab(t0 + 2 * BATCH, 0)
            fire_ab(t0 + 3 * BATCH, 1)

        wait_d(0)
        fire_e(t0, 0)
        wait_d(1)
        fire_e(t1, 1)

    wait_e(0)
    wait_e(1)


def kernel(idx, indexes, vectors, dims):
    del dims
    idx_t = jnp.swapaxes(idx, 0, 1).reshape(-1)
    vec_flat = vectors.reshape(DIM * KS, SUBDIM)
    mesh = plsc.VectorSubcoreMesh(core_axis_name="c", subcore_axis_name="s")
    cp = pltpu.CompilerParams(
        needs_layout_passes=False, use_tc_tiling_on_sc=False)
    decode = pl.kernel(
        _pq_decode_body,
        out_type=jax.ShapeDtypeStruct((N_TOK * DIM, SUBDIM), jnp.float32),
        mesh=mesh,
        scratch_types=[
            pltpu.VMEM((C,), jnp.int32),
            pltpu.VMEM((C,), jnp.int32),
            pltpu.VMEM((C, DIM), jnp.int32),
            pltpu.VMEM((C, DIM), jnp.int32),
            pltpu.VMEM((C * DIM,), jnp.int32),
            pltpu.VMEM((C * DIM,), jnp.int32),
            pltpu.VMEM((C * DIM, SUBDIM), jnp.float32),
            pltpu.VMEM((C * DIM, SUBDIM), jnp.float32),
            pltpu.VMEM_SHARED((DIM * KS, SUBDIM), jnp.float32),
            pltpu.SemaphoreType.DMA,
            pltpu.SemaphoreType.DMA,
            pltpu.SemaphoreType.DMA,
            pltpu.SemaphoreType.DMA,
            pltpu.SemaphoreType.DMA,
            pltpu.SemaphoreType.DMA,
        ],
        compiler_params=cp,
    )
    out = decode(idx_t, indexes, vec_flat)
    out = out.reshape(HIST, BATCH, ORIG)
    return jnp.swapaxes(out, 0, 1)


# speedup vs baseline: 4.3250x; 1.4664x over previous
import jax
import jax.numpy as jnp
from jax import lax
from jax.experimental import pallas as pl
from jax.experimental.pallas import tpu as pltpu
from jax.experimental.pallas import tpu_sc as plsc

VOCAB = 100000
DIM = 8
KS = 256
SUBDIM = 16
ORIG = DIM * SUBDIM

BATCH = 4096
HIST = 50
NC, NS, LANES = 2, 16, 16
NW = NC * NS
N_TOK = BATCH * HIST
C = 128
PAIRS = HIST // 2
VSHARD = VOCAB // NS


def _pq_decode_body(idx_hbm, indexes_hbm, vectors_hbm, out_hbm,
                    idx0, idx1, codes0, codes1, flat0, flat1, out0, out1,
                    ish0, ish1, ish2, ish3, ish4, ish5, ish6, ish7,
                    vec_sh,
                    semb0, semb1, semd0, semd1, seme0, seme1):
    sid = lax.axis_index("s")
    wid = sid * NC + lax.axis_index("c")
    base = wid * C
    idx_v = (idx0, idx1)
    codes_v = (codes0, codes1)
    flat_v = (flat0, flat1)
    out_v = (out0, out1)
    idx_sh = (ish0, ish1, ish2, ish3, ish4, ish5, ish6, ish7)
    semb = (semb0, semb1)
    semd = (semd0, semd1)
    seme = (seme0, seme1)

    for d in range(DIM):
        for h2 in range(2):
            @pl.when(sid == 2 * d + h2)
            def _():
                pltpu.sync_copy(
                    indexes_hbm.at[d, pl.ds(h2 * (VOCAB // 2), VOCAB // 2)],
                    idx_sh[d].at[pl.ds(h2 * (VOCAB // 2), VOCAB // 2)])
    vrows = (DIM * KS) // NS
    pltpu.sync_copy(vectors_hbm.at[pl.ds(sid * vrows, vrows)],
                    vec_sh.at[pl.ds(sid * vrows, vrows)])
    plsc.subcore_barrier()

    iot = lax.broadcasted_iota(jnp.int32, (16,), 0)
    row_pat = iot >> 3
    col_pat = iot & 7
    off_pat = col_pat << 8

    def fire_ab(t0, b):
        pltpu.sync_copy(idx_hbm.at[pl.ds(t0, C)], idx_v[b])
        for d in range(DIM):
            pltpu.async_copy(
                idx_sh[d].at[idx_v[b]], codes_v[b].at[d], semb[b])

    def wait_b(b):
        for d in range(DIM):
            pltpu.make_async_copy(
                idx_sh[d].at[idx_v[b]], codes_v[b].at[d], semb[b]).wait()

    def build_flat(b):
        @pl.loop(0, C // 2)
        def _grp(g):
            codes16 = plsc.load_gather(codes_v[b], [col_pat, row_pat + 2 * g])
            flat_v[b][pl.ds(16 * g, 16)] = codes16 + off_pat

    def fire_d(b):
        for j in range(DIM):
            pltpu.async_copy(
                vec_sh.at[flat_v[b].at[pl.ds(128 * j, 128)]],
                out_v[b].at[pl.ds(128 * j, 128)], semd[b])

    def wait_d(b):
        for j in range(DIM):
            pltpu.make_async_copy(
                vec_sh.at[flat_v[b].at[pl.ds(128 * j, 128)]],
                out_v[b].at[pl.ds(128 * j, 128)], semd[b]).wait()

    def fire_e(t0, b):
        pltpu.async_copy(
            out_v[b], out_hbm.at[pl.ds(t0 * DIM, C * DIM)], seme[b])

    def wait_e(b):
        pltpu.make_async_copy(
            out_v[b], out_hbm.at[pl.ds(0, C * DIM)], seme[b]).wait()

    fire_ab(base, 0)
    fire_ab(base + BATCH, 1)

    @pl.loop(0, PAIRS)
    def _pair(g):
        t0 = (2 * g) * BATCH + base
        t1 = t0 + BATCH

        wait_b(0)
        build_flat(0)

        @pl.when(g > 0)
        def _():
            wait_e(0)
        fire_d(0)

        wait_b(1)
        build_flat(1)

        @pl.when(g > 0)
        def _():
            wait_e(1)
        fire_d(1)

        @pl.when(g < PAIRS - 1)
        def _():
            fire_ab(t0 + 2 * BATCH, 0)
            fire_ab(t0 + 3 * BATCH, 1)

        wait_d(0)
        fire_e(t0, 0)
        wait_d(1)
        fire_e(t1, 1)

    wait_e(0)
    wait_e(1)


def kernel(idx, indexes, vectors, dims):
    del dims
    idx_t = jnp.swapaxes(idx, 0, 1).reshape(-1)
    indexes_t = jnp.swapaxes(indexes, 0, 1)
    vec_flat = vectors.reshape(DIM * KS, SUBDIM)
    mesh = plsc.VectorSubcoreMesh(core_axis_name="c", subcore_axis_name="s")
    cp = pltpu.CompilerParams(
        needs_layout_passes=False, use_tc_tiling_on_sc=False)
    decode = pl.kernel(
        _pq_decode_body,
        out_type=jax.ShapeDtypeStruct((N_TOK * DIM, SUBDIM), jnp.float32),
        mesh=mesh,
        scratch_types=[
            pltpu.VMEM((C,), jnp.int32),
            pltpu.VMEM((C,), jnp.int32),
            pltpu.VMEM((DIM, C), jnp.int32),
            pltpu.VMEM((DIM, C), jnp.int32),
            pltpu.VMEM((C * DIM,), jnp.int32),
            pltpu.VMEM((C * DIM,), jnp.int32),
            pltpu.VMEM((C * DIM, SUBDIM), jnp.float32),
            pltpu.VMEM((C * DIM, SUBDIM), jnp.float32),
        ] + [pltpu.VMEM_SHARED((VOCAB,), jnp.int32)] * DIM + [
            pltpu.VMEM_SHARED((DIM * KS, SUBDIM), jnp.float32),
            pltpu.SemaphoreType.DMA,
            pltpu.SemaphoreType.DMA,
            pltpu.SemaphoreType.DMA,
            pltpu.SemaphoreType.DMA,
            pltpu.SemaphoreType.DMA,
            pltpu.SemaphoreType.DMA,
        ],
        compiler_params=cp,
    )
    out = decode(idx_t, indexes_t, vec_flat)
    out = out.reshape(HIST, BATCH, ORIG)
    return jnp.swapaxes(out, 0, 1)


# speedup vs baseline: 4.5925x; 1.0619x over previous
import jax
import jax.numpy as jnp
from jax import lax
from jax.experimental import pallas as pl
from jax.experimental.pallas import tpu as pltpu
from jax.experimental.pallas import tpu_sc as plsc

VOCAB = 100000
DIM = 8
KS = 256
SUBDIM = 16
ORIG = DIM * SUBDIM

BATCH = 4096
HIST = 50
NC, NS, LANES = 2, 16, 16
NW = NC * NS
N_TOK = BATCH * HIST
C = 128
PAIRS = HIST // 2
VSHARD = VOCAB // NS


def _pq_decode_body(idx_hbm, indexes_hbm, vectors_hbm, out_hbm,
                    idx0, idx1, codes0, codes1, flat0, flat1, out0, out1,
                    ish0, ish1, ish2, ish3, ish4, ish5, ish6, ish7,
                    vec_sh,
                    semb0, semb1, semd0, semd1, seme0, seme1):
    sid = lax.axis_index("s")
    wid = sid * NC + lax.axis_index("c")
    base = wid * C
    idx_v = (idx0, idx1)
    codes_v = (codes0, codes1)
    flat_v = (flat0, flat1)
    out_v = (out0, out1)
    idx_sh = (ish0, ish1, ish2, ish3, ish4, ish5, ish6, ish7)
    semb = (semb0, semb1)
    semd = (semd0, semd1)
    seme = (seme0, seme1)

    for d in range(DIM):
        for h2 in range(2):
            @pl.when(sid == 2 * d + h2)
            def _():
                pltpu.sync_copy(
                    indexes_hbm.at[d, pl.ds(h2 * (VOCAB // 2), VOCAB // 2)],
                    idx_sh[d].at[pl.ds(h2 * (VOCAB // 2), VOCAB // 2)])
    vrows = (DIM * KS) // NS
    pltpu.sync_copy(vectors_hbm.at[pl.ds(sid * vrows, vrows)],
                    vec_sh.at[pl.ds(sid * vrows, vrows)])
    plsc.subcore_barrier()

    iot = lax.broadcasted_iota(jnp.int32, (16,), 0)
    row_pat = iot >> 3
    col_pat = iot & 7
    off_pat = col_pat << 8

    def fire_ab(t0, b):
        pltpu.sync_copy(idx_hbm.at[pl.ds(t0, C)], idx_v[b])
        for d in range(DIM):
            pltpu.async_copy(
                idx_sh[d].at[idx_v[b]], codes_v[b].at[d], semb[b])

    def wait_b(b):
        for d in range(DIM):
            pltpu.make_async_copy(
                idx_sh[d].at[idx_v[b]], codes_v[b].at[d], semb[b]).wait()

    def build_flat(b):
        @pl.loop(0, C // 2, step=2)
        def _grp(g):
            c0 = plsc.load_gather(codes_v[b], [col_pat, row_pat + 2 * g])
            c1 = plsc.load_gather(codes_v[b], [col_pat, row_pat + 2 * g + 2])
            flat_v[b][pl.ds(16 * g, 16)] = c0 + off_pat
            flat_v[b][pl.ds(16 * g + 16, 16)] = c1 + off_pat

    def fire_d(b):
        for j in range(DIM):
            pltpu.async_copy(
                vec_sh.at[flat_v[b].at[pl.ds(128 * j, 128)]],
                out_v[b].at[pl.ds(128 * j, 128)], semd[b])

    def wait_d(b):
        pltpu.make_async_copy(
            vec_sh.at[flat_v[b]], out_v[b], semd[b]).wait()

    def fire_e(t0, b):
        pltpu.async_copy(
            out_v[b], out_hbm.at[pl.ds(t0 * DIM, C * DIM)], seme[b])

    def wait_e(b):
        pltpu.make_async_copy(
            out_v[b], out_hbm.at[pl.ds(0, C * DIM)], seme[b]).wait()

    fire_ab(base, 0)
    fire_ab(base + BATCH, 1)

    @pl.loop(0, PAIRS)
    def _pair(g):
        t0 = (2 * g) * BATCH + base
        t1 = t0 + BATCH

        wait_b(0)
        build_flat(0)

        @pl.when(g > 0)
        def _():
            wait_e(0)
        fire_d(0)

        wait_b(1)
        build_flat(1)

        @pl.when(g > 0)
        def _():
            wait_e(1)
        fire_d(1)

        @pl.when(g < PAIRS - 1)
        def _():
            fire_ab(t0 + 2 * BATCH, 0)
            fire_ab(t0 + 3 * BATCH, 1)

        wait_d(0)
        fire_e(t0, 0)
        wait_d(1)
        fire_e(t1, 1)

    wait_e(0)
    wait_e(1)


def kernel(idx, indexes, vectors, dims):
    del dims
    idx_t = jnp.swapaxes(idx, 0, 1).reshape(-1)
    indexes_t = jnp.swapaxes(indexes, 0, 1)
    vec_flat = vectors.reshape(DIM * KS, SUBDIM)
    mesh = plsc.VectorSubcoreMesh(core_axis_name="c", subcore_axis_name="s")
    cp = pltpu.CompilerParams(
        needs_layout_passes=False, use_tc_tiling_on_sc=False)
    decode = pl.kernel(
        _pq_decode_body,
        out_type=jax.ShapeDtypeStruct((N_TOK * DIM, SUBDIM), jnp.float32),
        mesh=mesh,
        scratch_types=[
            pltpu.VMEM((C,), jnp.int32),
            pltpu.VMEM((C,), jnp.int32),
            pltpu.VMEM((DIM, C), jnp.int32),
            pltpu.VMEM((DIM, C), jnp.int32),
            pltpu.VMEM((C * DIM,), jnp.int32),
            pltpu.VMEM((C * DIM,), jnp.int32),
            pltpu.VMEM((C * DIM, SUBDIM), jnp.float32),
            pltpu.VMEM((C * DIM, SUBDIM), jnp.float32),
        ] + [pltpu.VMEM_SHARED((VOCAB,), jnp.int32)] * DIM + [
            pltpu.VMEM_SHARED((DIM * KS, SUBDIM), jnp.float32),
            pltpu.SemaphoreType.DMA,
            pltpu.SemaphoreType.DMA,
            pltpu.SemaphoreType.DMA,
            pltpu.SemaphoreType.DMA,
            pltpu.SemaphoreType.DMA,
            pltpu.SemaphoreType.DMA,
        ],
        compiler_params=cp,
    )
    out = decode(idx_t, indexes_t, vec_flat)
    out = out.reshape(HIST, BATCH, ORIG)
    return jnp.swapaxes(out, 0, 1)


# speedup vs baseline: 5.3035x; 1.1548x over previous
import jax
import jax.numpy as jnp
from jax import lax
from jax.experimental import pallas as pl
from jax.experimental.pallas import tpu as pltpu
from jax.experimental.pallas import tpu_sc as plsc

VOCAB = 100000
DIM = 8
KS = 256
SUBDIM = 16
ORIG = DIM * SUBDIM

BATCH = 4096
HIST = 50
NC, NS, LANES = 2, 16, 16
NW = NC * NS
N_TOK = BATCH * HIST
C = 128
PAIRS = HIST // 2
VSHARD = VOCAB // NS


def _pq_decode_body(idx_hbm, indexes_hbm, vectors_hbm, out_hbm,
                    idx0, idx1, codes0, codes1, flat0, flat1, out0, out1,
                    ish0, ish1, ish2, ish3, ish4, ish5, ish6, ish7,
                    vec_sh,
                    sema0, sema1, semb0, semb1, semd0, semd1, seme0, seme1):
    sid = lax.axis_index("s")
    wid = sid * NC + lax.axis_index("c")
    base = wid * C
    idx_v = (idx0, idx1)
    codes_v = (codes0, codes1)
    flat_v = (flat0, flat1)
    out_v = (out0, out1)
    idx_sh = (ish0, ish1, ish2, ish3, ish4, ish5, ish6, ish7)
    sema = (sema0, sema1)
    semb = (semb0, semb1)
    semd = (semd0, semd1)
    seme = (seme0, seme1)

    for d in range(DIM):
        for h2 in range(2):
            @pl.when(sid == 2 * d + h2)
            def _():
                pltpu.sync_copy(
                    indexes_hbm.at[d, pl.ds(h2 * (VOCAB // 2), VOCAB // 2)],
                    idx_sh[d].at[pl.ds(h2 * (VOCAB // 2), VOCAB // 2)])
    vrows = (DIM * KS) // NS
    pltpu.sync_copy(vectors_hbm.at[pl.ds(sid * vrows, vrows)],
                    vec_sh.at[pl.ds(sid * vrows, vrows)])
    plsc.subcore_barrier()

    iot = lax.broadcasted_iota(jnp.int32, (16,), 0)
    row_pat = iot >> 3
    col_pat = iot & 7
    off_pat = col_pat << 8

    def fire_a(t0, b):
        pltpu.async_copy(idx_hbm.at[pl.ds(t0, C)], idx_v[b], sema[b])

    def fire_b(b):
        pltpu.make_async_copy(
            idx_hbm.at[pl.ds(0, C)], idx_v[b], sema[b]).wait()
        for d in range(DIM):
            pltpu.async_copy(
                idx_sh[d].at[idx_v[b]], codes_v[b].at[d], semb[b])

    def wait_b(b):
        for d in range(DIM):
            pltpu.make_async_copy(
                idx_sh[d].at[idx_v[b]], codes_v[b].at[d], semb[b]).wait()

    def build_flat(b):
        @pl.loop(0, C // 2, step=2)
        def _grp(g):
            c0 = plsc.load_gather(codes_v[b], [col_pat, row_pat + 2 * g])
            c1 = plsc.load_gather(codes_v[b], [col_pat, row_pat + 2 * g + 2])
            flat_v[b][pl.ds(16 * g, 16)] = c0 + off_pat
            flat_v[b][pl.ds(16 * g + 16, 16)] = c1 + off_pat

    def fire_d(b):
        for j in range(DIM):
            pltpu.async_copy(
                vec_sh.at[flat_v[b].at[pl.ds(128 * j, 128)]],
                out_v[b].at[pl.ds(128 * j, 128)], semd[b])

    def wait_d(b):
        pltpu.make_async_copy(
            vec_sh.at[flat_v[b]], out_v[b], semd[b]).wait()

    def fire_e(t0, b):
        pltpu.async_copy(
            out_v[b], out_hbm.at[pl.ds(t0 * DIM, C * DIM)], seme[b])

    def wait_e(b):
        pltpu.make_async_copy(
            out_v[b], out_hbm.at[pl.ds(0, C * DIM)], seme[b]).wait()

    fire_a(base, 0)
    fire_a(base + BATCH, 1)
    fire_b(0)
    fire_b(1)

    @pl.loop(0, PAIRS)
    def _pair(g):
        t0 = (2 * g) * BATCH + base
        t1 = t0 + BATCH

        wait_b(0)
        @pl.when(g < PAIRS - 1)
        def _():
            fire_a(t0 + 2 * BATCH, 0)
        build_flat(0)

        @pl.when(g > 0)
        def _():
            wait_e(0)
        fire_d(0)

        wait_b(1)

        @pl.when(g < PAIRS - 1)
        def _():
            fire_a(t0 + 3 * BATCH, 1)
        build_flat(1)

        @pl.when(g > 0)
        def _():
            wait_e(1)
        fire_d(1)

        @pl.when(g < PAIRS - 1)
        def _():
            fire_b(0)
            fire_b(1)

        wait_d(0)
        fire_e(t0, 0)
        wait_d(1)
        fire_e(t1, 1)

    wait_e(0)
    wait_e(1)


def kernel(idx, indexes, vectors, dims):
    del dims
    idx_t = jnp.swapaxes(idx, 0, 1).reshape(-1)
    indexes_t = jnp.swapaxes(indexes, 0, 1)
    vec_flat = vectors.reshape(DIM * KS, SUBDIM)
    mesh = plsc.VectorSubcoreMesh(core_axis_name="c", subcore_axis_name="s")
    cp = pltpu.CompilerParams(
        needs_layout_passes=False, use_tc_tiling_on_sc=False)
    decode = pl.kernel(
        _pq_decode_body,
        out_type=jax.ShapeDtypeStruct((N_TOK * DIM, SUBDIM), jnp.float32),
        mesh=mesh,
        scratch_types=[
            pltpu.VMEM((C,), jnp.int32),
            pltpu.VMEM((C,), jnp.int32),
            pltpu.VMEM((DIM, C), jnp.int32),
            pltpu.VMEM((DIM, C), jnp.int32),
            pltpu.VMEM((C * DIM,), jnp.int32),
            pltpu.VMEM((C * DIM,), jnp.int32),
            pltpu.VMEM((C * DIM, SUBDIM), jnp.float32),
            pltpu.VMEM((C * DIM, SUBDIM), jnp.float32),
        ] + [pltpu.VMEM_SHARED((VOCAB,), jnp.int32)] * DIM + [
            pltpu.VMEM_SHARED((DIM * KS, SUBDIM), jnp.float32),
            pltpu.SemaphoreType.DMA,
            pltpu.SemaphoreType.DMA,
            pltpu.SemaphoreType.DMA,
            pltpu.SemaphoreType.DMA,
            pltpu.SemaphoreType.DMA,
            pltpu.SemaphoreType.DMA,
            pltpu.SemaphoreType.DMA,
            pltpu.SemaphoreType.DMA,
        ],
        compiler_params=cp,
    )
    out = decode(idx_t, indexes_t, vec_flat)
    out = out.reshape(HIST, BATCH, ORIG)
    return jnp.swapaxes(out, 0, 1)
